# baseline (speedup 1.0000x reference)
import functools

import jax
import jax.numpy as jnp
from jax import lax
from jax.experimental import pallas as pl
from jax.experimental.pallas import tpu as pltpu
from jax.experimental.pallas import tpu_sc as plsc

Y_VOCAB_ROWS = 1000000
HIDDEN = 32
NUM_CORES = 2
NUM_SUBCORES = 16
NW = NUM_CORES * NUM_SUBCORES

NVB = Y_VOCAB_ROWS // 128
VB_MAIN = (NVB // NW) & ~1
VB_REM = NVB - VB_MAIN * NW
VB_TAIL = Y_VOCAB_ROWS - NVB * 128


@jax.jit
def _rows_call(table_t, tail_lin):
    mesh = plsc.VectorSubcoreMesh(core_axis_name="c", subcore_axis_name="s")

    def body(tt_hbm, tail_hbm, out_hbm, tin0, tin1, t1a, t1b,
             hconst, dconst, si0, si1, so0, so1):
        wid = lax.axis_index("s") * NUM_CORES + lax.axis_index("c")
        iota16 = lax.iota(jnp.int32, 16)

        for d in range(16):
            for hh, h0 in enumerate((0, 16)):
                hv = h0 + ((iota16 + d) & 15)
                hconst[hh, d, :] = hv
                dconst[hh, d, :] = iota16 * HIDDEN + hv

        tins = (tin0, tin1)
        t1s = (t1a, t1b)
        sis = (si0, si1)
        sos = (so0, so1)

        def vb_of(k):
            return wid + NW * k

        def start_in(k, p):
            return pltpu.async_copy(
                tt_hbm.at[:, pl.ds(vb_of(k) * 128, 128)], tins[p], sis[p])

        def start_out(k, p):
            return pltpu.async_copy(
                t1s[p], out_hbm.at[pl.ds(vb_of(k) * 4096, 4096)], sos[p])

        def transpose_block(tin, t1):
            def d_loop(i, carry):
                for du in (0, 1):
                    d = 2 * i + du
                    for hh in (0, 1):
                        hvec = hconst[hh, d, :]
                        dvec = dconst[hh, d, :]
                        for c in range(8):
                            v = plsc.load_gather(tin, [hvec, c * 16 + iota16])
                            plsc.store_scatter(t1, [dvec + c * 512], v)
                return carry
            lax.fori_loop(0, 8, d_loop, 0)

        def wait_in(p):
            pltpu.make_async_copy(
                tt_hbm.at[:, pl.ds(0, 128)], tins[p], sis[p]).wait()

        def wait_out(p):
            pltpu.make_async_copy(
                t1s[p], out_hbm.at[pl.ds(0, 4096)], sos[p]).wait()

        start_in(0, 0)
        start_in(1, 1)

        def loop(i, carry):
            for p in (0, 1):
                k = 2 * i + p
                wait_in(p)

                @pl.when(i > 0)
                def _():
                    wait_out(p)

                transpose_block(tins[p], t1s[p])

                @pl.when(k + 2 < VB_MAIN)
                def _():
                    start_in(k + 2, p)

                start_out(k, p)
            return carry

        lax.fori_loop(0, VB_MAIN // 2, loop, 0)
        wait_out(0)
        wait_out(1)

        @pl.when(wid < VB_REM)
        def _():
            vb = NVB - VB_REM + wid
            pltpu.sync_copy(tt_hbm.at[:, pl.ds(vb * 128, 128)], tin0)
            transpose_block(tin0, t1a)
            pltpu.sync_copy(t1a, out_hbm.at[pl.ds(vb * 4096, 4096)])

        @pl.when(wid == VB_REM)
        def _():
            base = NVB * 128
            pltpu.sync_copy(tail_hbm, t1a.at[pl.ds(0, VB_TAIL * HIDDEN)])
            pltpu.sync_copy(t1a.at[pl.ds(0, VB_TAIL * HIDDEN)],
                            out_hbm.at[pl.ds(base * HIDDEN, VB_TAIL * HIDDEN)])

    run = pl.kernel(
        body,
        out_type=jax.ShapeDtypeStruct((Y_VOCAB_ROWS * HIDDEN,), jnp.float32),
        mesh=mesh,
        scratch_types=[
            pltpu.VMEM((HIDDEN, 128), jnp.float32),
            pltpu.VMEM((HIDDEN, 128), jnp.float32),
            pltpu.VMEM((4096,), jnp.float32),
            pltpu.VMEM((4096,), jnp.float32),
            pltpu.VMEM((2, 16, 16), jnp.int32),
            pltpu.VMEM((2, 16, 16), jnp.int32),
            pltpu.SemaphoreType.DMA,
            pltpu.SemaphoreType.DMA,
            pltpu.SemaphoreType.DMA,
            pltpu.SemaphoreType.DMA,
        ],
        compiler_params=pltpu.CompilerParams(
            use_tc_tiling_on_sc=True, needs_layout_passes=False),
    )
    return run(table_t, tail_lin)


PAIRS_PER_W = 25
BLOCK = 4 * 8 * 8 * 128


@jax.jit
def _gather_call(y4, table_lin):
    mesh = plsc.VectorSubcoreMesh(core_axis_name="c", subcore_axis_name="s")

    def body(y_hbm, tab_hbm, out_hbm,
             idx0, idx1, rows0, rows1, tout, cconst, dconst,
             si0, si1, sg0, sg1, st):
        wid = lax.axis_index("s") * NUM_CORES + lax.axis_index("c")
        iota16 = lax.iota(jnp.int32, 16)

        idxs = (idx0, idx1)
        rows = (rows0, rows1)
        sis = (si0, si1)
        sgs = (sg0, sg1)

        def pair_of(i):
            return wid * PAIRS_PER_W + i

        def start_idx(i, p):
            pr = pair_of(i)
            return pltpu.async_copy(
                y_hbm.at[pr >> 2, :, pr & 3, :], idxs[p], sis[p])

        def start_gather(p):
            hs = []
            for c in range(8):
                hs.append(pltpu.async_copy(
                    tab_hbm.at[idxs[p].at[c]],
                    rows[p].at[pl.ds(c * 128, 128)], sgs[p]))
            return hs

        def wait_gather(hs):
            for h in hs:
                h.wait()

        def start_store(i):
            return pltpu.async_copy(
                tout, out_hbm.at[pl.ds(pair_of(i) * BLOCK, BLOCK)], st)

        for d in range(16):
            for hh, h0 in enumerate((0, 16)):
                hv = h0 + ((iota16 + d) & 15)
                cconst[hh, d, :] = hv
                dconst[hh, d, :] = ((hv >> 3) << 13) + ((hv & 7) << 7) + iota16

        def transpose_pair(rv):
            def d_loop(d, carry):
                def g_loop(g, c2):
                    for hh in (0, 1):
                        cvec = cconst[hh, d, :]
                        dvec = dconst[hh, d, :]
                        def u_body(u):
                            b0 = g * 128 + u * 16
                            v = plsc.load_gather(rv, [b0 + iota16, cvec])
                            plsc.store_scatter(
                                tout, [dvec + (g * 1024 + u * 16)], v)
                        u_body(0); u_body(1); u_body(2); u_body(3)
                        u_body(4); u_body(5); u_body(6); u_body(7)
                    return c2
                lax.fori_loop(0, 8, g_loop, 0)
                return carry
            lax.fori_loop(0, 16, d_loop, 0)

        def wait_idx(p):
            pltpu.make_async_copy(y_hbm.at[0, :, 0, :], idxs[p], sis[p]).wait()

        def wait_gather(p):
            pltpu.make_async_copy(
                tab_hbm.at[pl.ds(0, 1024)], rows[p], sgs[p]).wait()

        def wait_store():
            pltpu.make_async_copy(tout, out_hbm.at[pl.ds(0, BLOCK)], st).wait()

        start_idx(0, 0)
        start_idx(1, 1)
        wait_idx(0)
        start_gather(0)

        def step(q, p):
            o = p ^ 1
            wait_gather(p)

            @pl.when(q + 2 < PAIRS_PER_W)
            def _():
                start_idx(q + 2, p)

            @pl.when(q + 1 < PAIRS_PER_W)
            def _():
                wait_idx(o)
                start_gather(o)

            @pl.when(q > 0)
            def _():
                wait_store()

            transpose_pair(rows[p])
            start_store(q)

        def loop(i, carry):
            step(2 * i, 0)
            step(2 * i + 1, 1)
            return carry

        lax.fori_loop(0, PAIRS_PER_W // 2, loop, 0)
        step(PAIRS_PER_W - 1, (PAIRS_PER_W - 1) & 1)
        wait_store()

    run = pl.kernel(
        body,
        out_type=jax.ShapeDtypeStruct((200 * 1024 * 4 * HIDDEN,), jnp.float32),
        mesh=mesh,
        scratch_types=[
            pltpu.VMEM((8, 128), jnp.int32),
            pltpu.VMEM((8, 128), jnp.int32),
            pltpu.VMEM((1024, HIDDEN), jnp.float32),
            pltpu.VMEM((1024, HIDDEN), jnp.float32),
            pltpu.VMEM((BLOCK,), jnp.float32),
            pltpu.VMEM((2, 16, 16), jnp.int32),
            pltpu.VMEM((2, 16, 16), jnp.int32),
            pltpu.SemaphoreType.DMA,
            pltpu.SemaphoreType.DMA,
            pltpu.SemaphoreType.DMA,
            pltpu.SemaphoreType.DMA,
            pltpu.SemaphoreType.DMA,
        ],
        compiler_params=pltpu.CompilerParams(
            use_tc_tiling_on_sc=False, needs_layout_passes=False),
    )
    return run(y4, table_lin.reshape(Y_VOCAB_ROWS, HIDDEN))


def kernel(y, table):
    S, BATCH, J = y.shape
    CB = BATCH // 128
    y4 = y.reshape(S, CB, 128, J).transpose(0, 1, 3, 2)
    tail_lin = lax.slice(table, (NVB * 128, 0), (Y_VOCAB_ROWS, HIDDEN)).reshape(-1)
    table_lin = _rows_call(table.T, tail_lin)
    out_flat = _gather_call(y4, table_lin)
    return (out_flat.reshape(S, J, 4, 8, 8, 128)
            .transpose(0, 3, 5, 1, 2, 4)
            .reshape(S, BATCH, J, HIDDEN))


# speedup vs baseline: 1.8696x; 1.8696x over previous
import functools

import jax
import jax.numpy as jnp
from jax import lax
from jax.experimental import pallas as pl
from jax.experimental.pallas import tpu as pltpu
from jax.experimental.pallas import tpu_sc as plsc

Y_VOCAB_ROWS = 1000000
HIDDEN = 32
NUM_CORES = 2
NUM_SUBCORES = 16
NW = NUM_CORES * NUM_SUBCORES

NVB = Y_VOCAB_ROWS // 128
VB_MAIN = (NVB // NW) & ~1
VB_REM = NVB - VB_MAIN * NW
VB_TAIL = Y_VOCAB_ROWS - NVB * 128


@jax.jit
def _rows_call(table_t, tail_lin):
    mesh = plsc.VectorSubcoreMesh(core_axis_name="c", subcore_axis_name="s")

    def body(tt_hbm, tail_hbm, out_hbm, tin0, tin1, t1a, t1b,
             hconst, dconst, si0, si1, so0, so1):
        wid = lax.axis_index("s") * NUM_CORES + lax.axis_index("c")
        iota16 = lax.iota(jnp.int32, 16)

        for d in range(16):
            for hh, h0 in enumerate((0, 16)):
                hv = h0 + ((iota16 + d) & 15)
                hconst[hh, d, :] = hv
                dconst[hh, d, :] = iota16 * HIDDEN + hv

        tins = (tin0, tin1)
        t1s = (t1a, t1b)
        sis = (si0, si1)
        sos = (so0, so1)

        def vb_of(k):
            return wid + NW * k

        def start_in(k, p):
            return pltpu.async_copy(
                tt_hbm.at[:, pl.ds(vb_of(k) * 128, 128)], tins[p], sis[p])

        def start_out(k, p):
            return pltpu.async_copy(
                t1s[p], out_hbm.at[pl.ds(vb_of(k) * 4096, 4096)], sos[p])

        def transpose_block(tin, t1):
            def d_loop(d, carry):
                for hh in (0, 1):
                    hvec = hconst[hh, d, :]
                    dvec = dconst[hh, d, :]

                    @plsc.parallel_loop(0, 8, unroll=8)
                    def _(c):
                        v = plsc.load_gather(tin, [hvec, c * 16 + iota16])
                        plsc.store_scatter(t1, [dvec + c * 512], v)
                return carry
            lax.fori_loop(0, 16, d_loop, 0)

        def wait_in(p):
            pltpu.make_async_copy(
                tt_hbm.at[:, pl.ds(0, 128)], tins[p], sis[p]).wait()

        def wait_out(p):
            pltpu.make_async_copy(
                t1s[p], out_hbm.at[pl.ds(0, 4096)], sos[p]).wait()

        start_in(0, 0)
        start_in(1, 1)

        def loop(i, carry):
            for p in (0, 1):
                k = 2 * i + p
                wait_in(p)

                @pl.when(i > 0)
                def _():
                    wait_out(p)

                transpose_block(tins[p], t1s[p])

                @pl.when(k + 2 < VB_MAIN)
                def _():
                    start_in(k + 2, p)

                start_out(k, p)
            return carry

        lax.fori_loop(0, VB_MAIN // 2, loop, 0)
        wait_out(0)
        wait_out(1)

        @pl.when(wid < VB_REM)
        def _():
            vb = NVB - VB_REM + wid
            pltpu.sync_copy(tt_hbm.at[:, pl.ds(vb * 128, 128)], tin0)
            transpose_block(tin0, t1a)
            pltpu.sync_copy(t1a, out_hbm.at[pl.ds(vb * 4096, 4096)])

        @pl.when(wid == VB_REM)
        def _():
            base = NVB * 128
            pltpu.sync_copy(tail_hbm, t1a.at[pl.ds(0, VB_TAIL * HIDDEN)])
            pltpu.sync_copy(t1a.at[pl.ds(0, VB_TAIL * HIDDEN)],
                            out_hbm.at[pl.ds(base * HIDDEN, VB_TAIL * HIDDEN)])

    run = pl.kernel(
        body,
        out_type=jax.ShapeDtypeStruct((Y_VOCAB_ROWS * HIDDEN,), jnp.float32),
        mesh=mesh,
        scratch_types=[
            pltpu.VMEM((HIDDEN, 128), jnp.float32),
            pltpu.VMEM((HIDDEN, 128), jnp.float32),
            pltpu.VMEM((4096,), jnp.float32),
            pltpu.VMEM((4096,), jnp.float32),
            pltpu.VMEM((2, 16, 16), jnp.int32),
            pltpu.VMEM((2, 16, 16), jnp.int32),
            pltpu.SemaphoreType.DMA,
            pltpu.SemaphoreType.DMA,
            pltpu.SemaphoreType.DMA,
            pltpu.SemaphoreType.DMA,
        ],
        compiler_params=pltpu.CompilerParams(
            use_tc_tiling_on_sc=True, needs_layout_passes=False),
    )
    return run(table_t, tail_lin)


PAIRS_PER_W = 25
BLOCK = 4 * 8 * 8 * 128


@jax.jit
def _gather_call(y4, table_lin):
    mesh = plsc.VectorSubcoreMesh(core_axis_name="c", subcore_axis_name="s")

    def body(y_hbm, tab_hbm, out_hbm,
             idx0, idx1, rows0, rows1, tout, cconst, dconst,
             si0, si1, sg0, sg1, st):
        wid = lax.axis_index("s") * NUM_CORES + lax.axis_index("c")
        iota16 = lax.iota(jnp.int32, 16)

        idxs = (idx0, idx1)
        rows = (rows0, rows1)
        sis = (si0, si1)
        sgs = (sg0, sg1)

        def pair_of(i):
            return wid * PAIRS_PER_W + i

        def start_idx(i, p):
            pr = pair_of(i)
            return pltpu.async_copy(
                y_hbm.at[pr >> 2, :, pr & 3, :], idxs[p], sis[p])

        def start_gather(p):
            hs = []
            for c in range(8):
                hs.append(pltpu.async_copy(
                    tab_hbm.at[idxs[p].at[c]],
                    rows[p].at[pl.ds(c * 128, 128)], sgs[p]))
            return hs

        def wait_gather(hs):
            for h in hs:
                h.wait()

        def start_store(i):
            return pltpu.async_copy(
                tout, out_hbm.at[pl.ds(pair_of(i) * BLOCK, BLOCK)], st)

        for d in range(16):
            for hh, h0 in enumerate((0, 16)):
                hv = h0 + ((iota16 + d) & 15)
                cconst[hh, d, :] = hv
                dconst[hh, d, :] = ((hv >> 3) << 13) + ((hv & 7) << 7) + iota16

        def transpose_pair(rv):
            def d_loop(d, carry):
                for hh in (0, 1):
                    cvec = cconst[hh, d, :]
                    dvec = dconst[hh, d, :]

                    @plsc.parallel_loop(0, 64, unroll=8)
                    def _(k):
                        v = plsc.load_gather(rv, [k * 16 + iota16, cvec])
                        plsc.store_scatter(
                            tout,
                            [dvec + (((k >> 3) << 10) + ((k & 7) << 4))], v)
                return carry
            lax.fori_loop(0, 16, d_loop, 0)

        def wait_idx(p):
            pltpu.make_async_copy(y_hbm.at[0, :, 0, :], idxs[p], sis[p]).wait()

        def wait_gather(p):
            pltpu.make_async_copy(
                tab_hbm.at[pl.ds(0, 1024)], rows[p], sgs[p]).wait()

        def wait_store():
            pltpu.make_async_copy(tout, out_hbm.at[pl.ds(0, BLOCK)], st).wait()

        start_idx(0, 0)
        start_idx(1, 1)
        wait_idx(0)
        start_gather(0)

        def step(q, p):
            o = p ^ 1
            wait_gather(p)

            @pl.when(q + 2 < PAIRS_PER_W)
            def _():
                start_idx(q + 2, p)

            @pl.when(q + 1 < PAIRS_PER_W)
            def _():
                wait_idx(o)
                start_gather(o)

            @pl.when(q > 0)
            def _():
                wait_store()

            transpose_pair(rows[p])
            start_store(q)

        def loop(i, carry):
            step(2 * i, 0)
            step(2 * i + 1, 1)
            return carry

        lax.fori_loop(0, PAIRS_PER_W // 2, loop, 0)
        step(PAIRS_PER_W - 1, (PAIRS_PER_W - 1) & 1)
        wait_store()

    run = pl.kernel(
        body,
        out_type=jax.ShapeDtypeStruct((200 * 1024 * 4 * HIDDEN,), jnp.float32),
        mesh=mesh,
        scratch_types=[
            pltpu.VMEM((8, 128), jnp.int32),
            pltpu.VMEM((8, 128), jnp.int32),
            pltpu.VMEM((1024, HIDDEN), jnp.float32),
            pltpu.VMEM((1024, HIDDEN), jnp.float32),
            pltpu.VMEM((BLOCK,), jnp.float32),
            pltpu.VMEM((2, 16, 16), jnp.int32),
            pltpu.VMEM((2, 16, 16), jnp.int32),
            pltpu.SemaphoreType.DMA,
            pltpu.SemaphoreType.DMA,
            pltpu.SemaphoreType.DMA,
            pltpu.SemaphoreType.DMA,
            pltpu.SemaphoreType.DMA,
        ],
        compiler_params=pltpu.CompilerParams(
            use_tc_tiling_on_sc=False, needs_layout_passes=False),
    )
    return run(y4, table_lin.reshape(Y_VOCAB_ROWS, HIDDEN))


def kernel(y, table):
    S, BATCH, J = y.shape
    CB = BATCH // 128
    y4 = y.reshape(S, CB, 128, J).transpose(0, 1, 3, 2)
    tail_lin = lax.slice(table, (NVB * 128, 0), (Y_VOCAB_ROWS, HIDDEN)).reshape(-1)
    table_lin = _rows_call(table.T, tail_lin)
    out_flat = _gather_call(y4, table_lin)
    return (out_flat.reshape(S, J, 4, 8, 8, 128)
            .transpose(0, 3, 5, 1, 2, 4)
            .reshape(S, BATCH, J, HIDDEN))


# speedup vs baseline: 2.3902x; 1.2785x over previous
import functools

import jax
import jax.numpy as jnp
from jax import lax
from jax.experimental import pallas as pl
from jax.experimental.pallas import tpu as pltpu
from jax.experimental.pallas import tpu_sc as plsc

Y_VOCAB_ROWS = 1000000
HIDDEN = 32
NUM_CORES = 2
NUM_SUBCORES = 16
NW = NUM_CORES * NUM_SUBCORES

NVB = Y_VOCAB_ROWS // 128
VB_MAIN = (NVB // NW) & ~3
VB_REM = NVB - VB_MAIN * NW
VB_TAIL = Y_VOCAB_ROWS - NVB * 128


@jax.jit
def _rows_call(table_t, tail_lin):
    mesh = plsc.VectorSubcoreMesh(core_axis_name="c", subcore_axis_name="s")

    def body(tt_hbm, tail_hbm, out_hbm, tin0, tin1, tin2, tin3,
             t1a, t1b, t1c, t1d,
             hconst, dconst, si0, si1, si2, si3, so0, so1, so2, so3):
        wid = lax.axis_index("s") * NUM_CORES + lax.axis_index("c")
        iota16 = lax.iota(jnp.int32, 16)

        for d in range(16):
            for hh, h0 in enumerate((0, 16)):
                hv = h0 + ((iota16 + d) & 15)
                hconst[hh, d, :] = hv
                dconst[hh, d, :] = iota16 * HIDDEN + hv

        tins = (tin0, tin1, tin2, tin3)
        t1s = (t1a, t1b, t1c, t1d)
        sis = (si0, si1, si2, si3)
        sos = (so0, so1, so2, so3)

        def vb_of(k):
            return wid + NW * k

        def start_in(k, p):
            return pltpu.async_copy(
                tt_hbm.at[:, pl.ds(vb_of(k) * 128, 128)], tins[p], sis[p])

        def start_out(k, p):
            return pltpu.async_copy(
                t1s[p], out_hbm.at[pl.ds(vb_of(k) * 4096, 4096)], sos[p])

        def transpose_block(tin, t1):
            def d_loop(d, carry):
                for hh in (0, 1):
                    hvec = hconst[hh, d, :]
                    dvec = dconst[hh, d, :]

                    @plsc.parallel_loop(0, 8, unroll=8)
                    def _(c):
                        v = plsc.load_gather(tin, [hvec, c * 16 + iota16])
                        plsc.store_scatter(t1, [dvec + c * 512], v)
                return carry
            lax.fori_loop(0, 16, d_loop, 0)

        def wait_in(p):
            pltpu.make_async_copy(
                tt_hbm.at[:, pl.ds(0, 128)], tins[p], sis[p]).wait()

        def wait_out(p):
            pltpu.make_async_copy(
                t1s[p], out_hbm.at[pl.ds(0, 4096)], sos[p]).wait()

        for p in range(4):
            start_in(p, p)

        def loop(i, carry):
            for p in range(4):
                k = 4 * i + p
                wait_in(p)

                @pl.when(i > 0)
                def _():
                    wait_out(p)

                transpose_block(tins[p], t1s[p])

                @pl.when(k + 4 < VB_MAIN)
                def _():
                    start_in(k + 4, p)

                start_out(k, p)
            return carry

        lax.fori_loop(0, VB_MAIN // 4, loop, 0)
        for p in range(4):
            wait_out(p)

        @pl.when(wid < VB_REM)
        def _():
            vb = NVB - VB_REM + wid
            pltpu.sync_copy(tt_hbm.at[:, pl.ds(vb * 128, 128)], tin0)
            transpose_block(tin0, t1a)
            pltpu.sync_copy(t1a, out_hbm.at[pl.ds(vb * 4096, 4096)])

        @pl.when(wid == VB_REM)
        def _():
            base = NVB * 128
            pltpu.sync_copy(tail_hbm, t1a.at[pl.ds(0, VB_TAIL * HIDDEN)])
            pltpu.sync_copy(t1a.at[pl.ds(0, VB_TAIL * HIDDEN)],
                            out_hbm.at[pl.ds(base * HIDDEN, VB_TAIL * HIDDEN)])

    run = pl.kernel(
        body,
        out_type=jax.ShapeDtypeStruct((Y_VOCAB_ROWS * HIDDEN,), jnp.float32),
        mesh=mesh,
        scratch_types=[
            pltpu.VMEM((HIDDEN, 128), jnp.float32),
            pltpu.VMEM((HIDDEN, 128), jnp.float32),
            pltpu.VMEM((HIDDEN, 128), jnp.float32),
            pltpu.VMEM((HIDDEN, 128), jnp.float32),
            pltpu.VMEM((4096,), jnp.float32),
            pltpu.VMEM((4096,), jnp.float32),
            pltpu.VMEM((4096,), jnp.float32),
            pltpu.VMEM((4096,), jnp.float32),
            pltpu.VMEM((2, 16, 16), jnp.int32),
            pltpu.VMEM((2, 16, 16), jnp.int32),
            pltpu.SemaphoreType.DMA,
            pltpu.SemaphoreType.DMA,
            pltpu.SemaphoreType.DMA,
            pltpu.SemaphoreType.DMA,
            pltpu.SemaphoreType.DMA,
            pltpu.SemaphoreType.DMA,
            pltpu.SemaphoreType.DMA,
            pltpu.SemaphoreType.DMA,
        ],
        compiler_params=pltpu.CompilerParams(
            use_tc_tiling_on_sc=True, needs_layout_passes=False),
    )
    return run(table_t, tail_lin)


PAIRS_PER_W = 25
BLOCK = 4 * 8 * 8 * 128


@jax.jit
def _gather_call(y4, table_lin):
    mesh = plsc.VectorSubcoreMesh(core_axis_name="c", subcore_axis_name="s")

    def body(y_hbm, tab_hbm, out_hbm,
             idx0, idx1, rows0, rows1, tout, cconst, dconst,
             si0, si1, sg0, sg1, st):
        wid = lax.axis_index("s") * NUM_CORES + lax.axis_index("c")
        iota16 = lax.iota(jnp.int32, 16)

        idxs = (idx0, idx1)
        rows = (rows0, rows1)
        sis = (si0, si1)
        sgs = (sg0, sg1)

        def pair_of(i):
            return wid * PAIRS_PER_W + i

        def start_idx(i, p):
            pr = pair_of(i)
            return pltpu.async_copy(
                y_hbm.at[pr >> 2, :, pr & 3, :], idxs[p], sis[p])

        def start_gather(p):
            hs = []
            for c in range(8):
                hs.append(pltpu.async_copy(
                    tab_hbm.at[idxs[p].at[c]],
                    rows[p].at[pl.ds(c * 128, 128)], sgs[p]))
            return hs

        def wait_gather(hs):
            for h in hs:
                h.wait()

        def start_store(i):
            return pltpu.async_copy(
                tout, out_hbm.at[pl.ds(pair_of(i) * BLOCK, BLOCK)], st)

        for d in range(16):
            for hh, h0 in enumerate((0, 16)):
                hv = h0 + ((iota16 + d) & 15)
                cconst[hh, d, :] = hv
                dconst[hh, d, :] = ((hv >> 3) << 13) + ((hv & 7) << 7) + iota16

        def transpose_pair(rv):
            def d_loop(d, carry):
                for hh in (0, 1):
                    cvec = cconst[hh, d, :]
                    dvec = dconst[hh, d, :]

                    @plsc.parallel_loop(0, 64, unroll=8)
                    def _(k):
                        v = plsc.load_gather(rv, [k * 16 + iota16, cvec])
                        plsc.store_scatter(
                            tout,
                            [dvec + (((k >> 3) << 10) + ((k & 7) << 4))], v)
                return carry
            lax.fori_loop(0, 16, d_loop, 0)

        def wait_idx(p):
            pltpu.make_async_copy(y_hbm.at[0, :, 0, :], idxs[p], sis[p]).wait()

        def wait_gather(p):
            pltpu.make_async_copy(
                tab_hbm.at[pl.ds(0, 1024)], rows[p], sgs[p]).wait()

        def wait_store():
            pltpu.make_async_copy(tout, out_hbm.at[pl.ds(0, BLOCK)], st).wait()

        start_idx(0, 0)
        start_idx(1, 1)
        wait_idx(0)
        start_gather(0)

        def step(q, p):
            o = p ^ 1
            wait_gather(p)

            @pl.when(q + 2 < PAIRS_PER_W)
            def _():
                start_idx(q + 2, p)

            @pl.when(q + 1 < PAIRS_PER_W)
            def _():
                wait_idx(o)
                start_gather(o)

            @pl.when(q > 0)
            def _():
                wait_store()

            transpose_pair(rows[p])
            start_store(q)

        def loop(i, carry):
            step(2 * i, 0)
            step(2 * i + 1, 1)
            return carry

        lax.fori_loop(0, PAIRS_PER_W // 2, loop, 0)
        step(PAIRS_PER_W - 1, (PAIRS_PER_W - 1) & 1)
        wait_store()

    run = pl.kernel(
        body,
        out_type=jax.ShapeDtypeStruct((200 * 1024 * 4 * HIDDEN,), jnp.float32),
        mesh=mesh,
        scratch_types=[
            pltpu.VMEM((8, 128), jnp.int32),
            pltpu.VMEM((8, 128), jnp.int32),
            pltpu.VMEM((1024, HIDDEN), jnp.float32),
            pltpu.VMEM((1024, HIDDEN), jnp.float32),
            pltpu.VMEM((BLOCK,), jnp.float32),
            pltpu.VMEM((2, 16, 16), jnp.int32),
            pltpu.VMEM((2, 16, 16), jnp.int32),
            pltpu.SemaphoreType.DMA,
            pltpu.SemaphoreType.DMA,
            pltpu.SemaphoreType.DMA,
            pltpu.SemaphoreType.DMA,
            pltpu.SemaphoreType.DMA,
        ],
        compiler_params=pltpu.CompilerParams(
            use_tc_tiling_on_sc=False, needs_layout_passes=False),
    )
    return run(y4, table_lin.reshape(Y_VOCAB_ROWS, HIDDEN))


def kernel(y, table):
    S, BATCH, J = y.shape
    CB = BATCH // 128
    y4 = y.reshape(S, CB, 128, J).transpose(0, 1, 3, 2)
    tail_lin = lax.slice(table, (NVB * 128, 0), (Y_VOCAB_ROWS, HIDDEN)).reshape(-1)
    table_lin = _rows_call(table.T, tail_lin)
    out_flat = _gather_call(y4, table_lin)
    return (out_flat.reshape(S, J, 4, 8, 8, 128)
            .transpose(0, 3, 5, 1, 2, 4)
            .reshape(S, BATCH, J, HIDDEN))
